# baseline (speedup 1.0000x reference)
import functools

import jax
import jax.numpy as jnp
from jax import lax
from jax.experimental import pallas as pl
from jax.experimental.pallas import tpu as pltpu
from jax.experimental.pallas import tpu_sc as plsc

_BATCH = 16384
_FIELDS = 26
_L2 = 1e-06

_NC = 2
_NS = 16
_BPS = _BATCH // _NS
_CHUNK = _FIELDS * _BPS
_LANES = 16
_TBL = 1000000
_HALF = _TBL // 2
_HOFF = _HALF - 8
_STRIPE = 31488
_HALFP = _STRIPE * _NS


def _sc_gather_sum(idx_ref, wh_ref, xw_ref, idx_v, vals_v, acc_v, stage_v,
                   w_sh, sem):
  cid = lax.axis_index("c")
  sid = lax.axis_index("s")
  pltpu.sync_copy(idx_ref.at[sid], idx_v)
  off = sid * _STRIPE
  pltpu.sync_copy(wh_ref.at[cid, pl.ds(off, _STRIPE)], stage_v)
  pltpu.sync_copy(stage_v, w_sh.at[pl.ds(off, _STRIPE)])
  clamp0 = jnp.full((_LANES,), _HALF, jnp.int32)
  sub1 = jnp.full((_LANES,), _HOFF, jnp.int32)
  zero = jnp.zeros((_LANES,), jnp.int32)
  @pl.when(cid == 0)
  def _():
    def body(i, carry):
      s = i * _LANES
      idx_v[pl.ds(s, _LANES)] = jnp.minimum(idx_v[pl.ds(s, _LANES)], clamp0)
      return carry
    lax.fori_loop(0, _CHUNK // _LANES, body, 0)
  @pl.when(cid == 1)
  def _():
    def body(i, carry):
      s = i * _LANES
      idx_v[pl.ds(s, _LANES)] = jnp.maximum(
          idx_v[pl.ds(s, _LANES)] - sub1, zero)
      return carry
    lax.fori_loop(0, _CHUNK // _LANES, body, 0)
  plsc.subcore_barrier()
  pltpu.async_copy(w_sh.at[idx_v], vals_v, sem).wait()
  def sum_body(g, carry):
    s = g * _LANES
    acc = vals_v[pl.ds(s, _LANES)]
    for f in range(1, _FIELDS):
      acc = acc + vals_v[pl.ds(s + f * _BPS, _LANES)]
    acc_v[pl.ds(s, _LANES)] = acc
    return carry
  lax.fori_loop(0, _BPS // _LANES, sum_body, 0)
  pltpu.sync_copy(acc_v, xw_ref.at[cid, pl.ds(sid * _BPS, _BPS)])


@jax.jit
def _sc_xw(idx_arranged, w_halves):
  mesh = plsc.VectorSubcoreMesh(core_axis_name="c", subcore_axis_name="s")
  return pl.kernel(
      _sc_gather_sum,
      out_type=jax.ShapeDtypeStruct((_NC, _BATCH), jnp.float32),
      mesh=mesh,
      scratch_types=[
          pltpu.VMEM((_CHUNK,), jnp.int32),
          pltpu.VMEM((_CHUNK,), jnp.float32),
          pltpu.VMEM((_BPS,), jnp.float32),
          pltpu.VMEM((_STRIPE,), jnp.float32),
          pltpu.VMEM_SHARED((_HALFP,), jnp.float32),
          pltpu.SemaphoreType.DMA,
      ],
  )(idx_arranged, w_halves)


def _tc_head(xw2_ref, y_ref, b_ref, yprob_ref, loss_ref):
  xw = xw2_ref[0] + xw2_ref[1]
  logits = xw + b_ref[0]
  yprob_ref[...] = 1.0 / (1.0 + jnp.exp(-logits))
  bce = (jnp.maximum(logits, 0.0) - logits * y_ref[...]
         + jnp.log1p(jnp.exp(-jnp.abs(logits))))
  loss_ref[0] = (jnp.sum(bce) / _BATCH) + _L2 * 0.5 * jnp.sum(xw * xw)


@jax.jit
def _tc_loss(xw2, y, b):
  yprob, loss = pl.pallas_call(
      _tc_head,
      out_shape=(
          jax.ShapeDtypeStruct((128, 128), jnp.float32),
          jax.ShapeDtypeStruct((1,), jnp.float32),
      ),
      in_specs=[
          pl.BlockSpec(memory_space=pltpu.VMEM),
          pl.BlockSpec(memory_space=pltpu.VMEM),
          pl.BlockSpec(memory_space=pltpu.SMEM),
      ],
      out_specs=(
          pl.BlockSpec(memory_space=pltpu.VMEM),
          pl.BlockSpec(memory_space=pltpu.SMEM),
      ),
  )(xw2.reshape(_NC, 128, 128), y.reshape(128, 128), b)
  return yprob.reshape(-1), loss[0]


def kernel(indices, y, w, b):
  idx = indices.astype(jnp.int32)
  idx_arranged = (
      idx.reshape(_NS, _BPS, _FIELDS).transpose(0, 2, 1).reshape(_NS, _CHUNK)
  )
  w1 = w.reshape(-1)
  zeros8 = jnp.zeros((8,), jnp.float32)
  pad0 = jnp.zeros((_HALFP - _HALF,), jnp.float32)
  pad1 = jnp.zeros((_HALFP - _HALF - 8,), jnp.float32)
  w_halves = jnp.stack([
      jnp.concatenate([w1[:_HALF], pad0]),
      jnp.concatenate([zeros8, w1[_HALF:], pad1]),
  ])
  xw2 = _sc_xw(idx_arranged, w_halves)
  return _tc_loss(xw2, y, b)


# speedup vs baseline: 2.8789x; 2.8789x over previous
import functools

import jax
import jax.numpy as jnp
from jax import lax
from jax.experimental import pallas as pl
from jax.experimental.pallas import tpu as pltpu
from jax.experimental.pallas import tpu_sc as plsc

_BATCH = 16384
_FIELDS = 26
_L2 = 1e-06

_NC = 2
_NS = 16
_NW = _NC * _NS
_BPW = _BATCH // _NW
_CHUNK = _FIELDS * _BPW
_LANES = 16
_NSTREAM = 4
_QS = _CHUNK // _NSTREAM


def _sc_gather_sum(idx_ref, w_ref, xw_ref, idx_v, vals_v, acc_v, sem):
  wid = lax.axis_index("s") * _NC + lax.axis_index("c")
  pltpu.sync_copy(idx_ref.at[wid], idx_v)
  copies = [
      pltpu.async_copy(
          w_ref.at[idx_v.at[pl.ds(k * _QS, _QS)]],
          vals_v.at[pl.ds(k * _QS, _QS)],
          sem,
      )
      for k in range(_NSTREAM)
  ]
  for cp in copies:
    cp.wait()
  for g in range(_BPW // _LANES):
    acc = vals_v[pl.ds(g * _LANES, _LANES)]
    for f in range(1, _FIELDS):
      acc = acc + vals_v[pl.ds(f * _BPW + g * _LANES, _LANES)]
    acc_v[pl.ds(g * _LANES, _LANES)] = acc
  pltpu.sync_copy(acc_v, xw_ref.at[pl.ds(wid * _BPW, _BPW)])


@jax.jit
def _sc_xw(idx_arranged, w1d):
  mesh = plsc.VectorSubcoreMesh(core_axis_name="c", subcore_axis_name="s")
  return pl.kernel(
      _sc_gather_sum,
      out_type=jax.ShapeDtypeStruct((_BATCH,), jnp.float32),
      mesh=mesh,
      scratch_types=[
          pltpu.VMEM((_CHUNK,), jnp.int32),
          pltpu.VMEM((_CHUNK,), jnp.float32),
          pltpu.VMEM((_BPW,), jnp.float32),
          pltpu.SemaphoreType.DMA,
      ],
  )(idx_arranged, w1d)


def _tc_head(xw_ref, y_ref, b_ref, yprob_ref, loss_ref):
  xw = xw_ref[...]
  logits = xw + b_ref[0]
  yprob_ref[...] = 1.0 / (1.0 + jnp.exp(-logits))
  bce = (jnp.maximum(logits, 0.0) - logits * y_ref[...]
         + jnp.log1p(jnp.exp(-jnp.abs(logits))))
  loss_ref[0] = (jnp.sum(bce) / _BATCH) + _L2 * 0.5 * jnp.sum(xw * xw)


@jax.jit
def _tc_loss(xw, y, b):
  yprob, loss = pl.pallas_call(
      _tc_head,
      out_shape=(
          jax.ShapeDtypeStruct((128, 128), jnp.float32),
          jax.ShapeDtypeStruct((1,), jnp.float32),
      ),
      in_specs=[
          pl.BlockSpec(memory_space=pltpu.VMEM),
          pl.BlockSpec(memory_space=pltpu.VMEM),
          pl.BlockSpec(memory_space=pltpu.SMEM),
      ],
      out_specs=(
          pl.BlockSpec(memory_space=pltpu.VMEM),
          pl.BlockSpec(memory_space=pltpu.SMEM),
      ),
  )(xw.reshape(128, 128), y.reshape(128, 128), b)
  return yprob.reshape(-1), loss[0]


def kernel(indices, y, w, b):
  idx = indices.astype(jnp.int32)
  idx_arranged = (
      idx.reshape(_NW, _BPW, _FIELDS).transpose(0, 2, 1).reshape(_NW, _CHUNK)
  )
  xw = _sc_xw(idx_arranged, w.reshape(-1))
  return _tc_loss(xw, y, b)
